# baseline (speedup 1.0000x reference)
import functools

import jax
import jax.numpy as jnp
from jax import lax
from jax.experimental import pallas as pl
from jax.experimental.pallas import tpu as pltpu
from jax.experimental.pallas import tpu_sc as plsc

N_NODES = 10000
N_EDGES = 160000
F_IN = 256
F_HID = 128

NCORE = 2
NSUB = 16
NW = NCORE * NSUB
CHUNK = 64
EPW = 5184
NCHUNK = EPW // CHUNK
E_PAD = EPW * NW
N_ACC = 10112
ROWS_ACC = N_ACC // NSUB
ROW_BLK = 1000
NBUF = 3
PDEPTH = 2


def _vmesh():
    return plsc.VectorSubcoreMesh(core_axis_name="c", subcore_axis_name="s")


def _sc_deg(dstp, ones128, zeros128):
    @functools.partial(
        pl.kernel,
        out_type=jax.ShapeDtypeStruct((NCORE, N_ACC, F_HID), jnp.float32),
        mesh=_vmesh(),
        scratch_types=[
            pltpu.VMEM((NCHUNK, CHUNK), jnp.int32),
            pltpu.VMEM((CHUNK, F_HID), jnp.float32),
            pltpu.VMEM_SHARED((N_ACC, F_HID), jnp.float32),
        ],
    )
    def k(dst_hbm, ones_hbm, zeros_hbm, deg_hbm, dst_v, ones_v, acc_sh):
        c = lax.axis_index("c")
        s = lax.axis_index("s")
        w = c * NSUB + s
        pltpu.sync_copy(dst_hbm.at[w], dst_v)
        pltpu.sync_copy(ones_hbm, ones_v)
        pltpu.sync_copy(zeros_hbm.at[pl.ds(s * ROWS_ACC, ROWS_ACC)],
                        acc_sh.at[pl.ds(s * ROWS_ACC, ROWS_ACC)])
        plsc.subcore_barrier()

        @pl.loop(0, NCHUNK)
        def _(j):
            pltpu.sync_copy(ones_v, acc_sh.at[dst_v.at[j]], add=True)

        plsc.subcore_barrier()
        pltpu.sync_copy(acc_sh.at[pl.ds(s * ROWS_ACC, ROWS_ACC)],
                        deg_hbm.at[c, pl.ds(s * ROWS_ACC, ROWS_ACC)])

    return k(dstp, ones128, zeros128)


def _sc_scatter(z, srcp, dstp, zeros128):
    @functools.partial(
        pl.kernel,
        out_type=jax.ShapeDtypeStruct((NCORE, N_ACC, F_HID), jnp.float32),
        mesh=_vmesh(),
        scratch_types=[
            pltpu.VMEM((NCHUNK, CHUNK), jnp.int32),
            pltpu.VMEM((NCHUNK, CHUNK), jnp.int32),
            pltpu.VMEM((NBUF, CHUNK, F_HID), jnp.float32),
            pltpu.VMEM_SHARED((N_ACC, F_HID), jnp.float32),
            pltpu.SemaphoreType.DMA((NBUF,)),
            pltpu.SemaphoreType.DMA((NBUF,)),
        ],
    )
    def k(z_hbm, src_hbm, dst_hbm, zeros_hbm, out_hbm,
          src_v, dst_v, buf, acc_sh, gsem, ssem):
        c = lax.axis_index("c")
        s = lax.axis_index("s")
        w = c * NSUB + s
        pltpu.sync_copy(src_hbm.at[w], src_v)
        pltpu.sync_copy(dst_hbm.at[w], dst_v)
        pltpu.sync_copy(zeros_hbm.at[pl.ds(s * ROWS_ACC, ROWS_ACC)],
                        acc_sh.at[pl.ds(s * ROWS_ACC, ROWS_ACC)])
        plsc.subcore_barrier()

        for b in range(PDEPTH):
            pltpu.async_copy(z_hbm.at[src_v.at[b]], buf.at[b], gsem.at[b])

        @pl.loop(0, NCHUNK, step=NBUF)
        def _(j0):
            for b in range(NBUF):
                j = j0 + b
                pltpu.make_async_copy(z_hbm.at[src_v.at[j]], buf.at[b],
                                      gsem.at[b]).wait()
                pltpu.async_copy(buf.at[b], acc_sh.at[dst_v.at[j]],
                                 ssem.at[b], add=True)
                bn = (b + PDEPTH) % NBUF

                @pl.when(j + PDEPTH - NBUF >= 0)
                def _():
                    pltpu.make_async_copy(buf.at[bn],
                                          acc_sh.at[dst_v.at[j]],
                                          ssem.at[bn]).wait()

                @pl.when(j + PDEPTH < NCHUNK)
                def _():
                    pltpu.async_copy(z_hbm.at[src_v.at[j + PDEPTH]],
                                     buf.at[bn], gsem.at[bn])

        for k in range(NCHUNK - (NBUF - PDEPTH), NCHUNK):
            bb = k % NBUF
            pltpu.make_async_copy(buf.at[bb], acc_sh.at[dst_v.at[0]],
                                  ssem.at[bb]).wait()

        plsc.subcore_barrier()
        pltpu.sync_copy(acc_sh.at[pl.ds(s * ROWS_ACC, ROWS_ACC)],
                        out_hbm.at[c, pl.ds(s * ROWS_ACC, ROWS_ACC)])

    return k(z, srcp, dstp, zeros128)


def _dinv_block(d_ref):
    d = d_ref[0][:, 0:1] + d_ref[1][:, 0:1] + 1.0
    return lax.rsqrt(d)


def _tc_lin1(x, W1, deg):
    def body(x_ref, w_ref, d_ref, o_ref):
        y = jnp.dot(x_ref[...], w_ref[...], preferred_element_type=jnp.float32)
        o_ref[...] = y * _dinv_block(d_ref)

    return pl.pallas_call(
        body,
        grid=(N_NODES // ROW_BLK,),
        in_specs=[
            pl.BlockSpec((ROW_BLK, F_IN), lambda i: (i, 0)),
            pl.BlockSpec((F_IN, F_HID), lambda i: (0, 0)),
            pl.BlockSpec((NCORE, ROW_BLK, F_HID), lambda i: (0, i, 0)),
        ],
        out_specs=pl.BlockSpec((ROW_BLK, F_HID), lambda i: (i, 0)),
        out_shape=jax.ShapeDtypeStruct((N_NODES, F_HID), jnp.float32),
    )(x, W1, deg)


def _tc_lin2(acc, z1, deg, b1, W3):
    def body(a_ref, z_ref, d_ref, b_ref, w_ref, o_ref):
        dinv = _dinv_block(d_ref)
        h = (a_ref[0] + a_ref[1] + z_ref[...]) * dinv + b_ref[...]
        h = jnp.maximum(h, 0.0)
        y = jnp.dot(h, w_ref[...], preferred_element_type=jnp.float32)
        o_ref[...] = y * dinv

    return pl.pallas_call(
        body,
        grid=(N_NODES // ROW_BLK,),
        in_specs=[
            pl.BlockSpec((NCORE, ROW_BLK, F_HID), lambda i: (0, i, 0)),
            pl.BlockSpec((ROW_BLK, F_HID), lambda i: (i, 0)),
            pl.BlockSpec((NCORE, ROW_BLK, F_HID), lambda i: (0, i, 0)),
            pl.BlockSpec((1, F_HID), lambda i: (0, 0)),
            pl.BlockSpec((F_HID, F_HID), lambda i: (0, 0)),
        ],
        out_specs=pl.BlockSpec((ROW_BLK, F_HID), lambda i: (i, 0)),
        out_shape=jax.ShapeDtypeStruct((N_NODES, F_HID), jnp.float32),
    )(acc, z1, deg, b1, W3)


def _tc_final(acc, z2, deg, b3):
    def body(a_ref, z_ref, d_ref, b_ref, o_ref):
        dinv = _dinv_block(d_ref)
        o_ref[...] = (a_ref[0] + a_ref[1] + z_ref[...]) * dinv + b_ref[...]

    return pl.pallas_call(
        body,
        grid=(N_NODES // ROW_BLK,),
        in_specs=[
            pl.BlockSpec((NCORE, ROW_BLK, F_HID), lambda i: (0, i, 0)),
            pl.BlockSpec((ROW_BLK, F_HID), lambda i: (i, 0)),
            pl.BlockSpec((NCORE, ROW_BLK, F_HID), lambda i: (0, i, 0)),
            pl.BlockSpec((1, F_HID), lambda i: (0, 0)),
        ],
        out_specs=pl.BlockSpec((ROW_BLK, F_HID), lambda i: (i, 0)),
        out_shape=jax.ShapeDtypeStruct((N_NODES, F_HID), jnp.float32),
    )(acc, z2, deg, b3)


def kernel(x, edge_index, W1, b1, W3, b3):
    src = edge_index[0]
    dst = edge_index[1]
    pad_src = jnp.zeros((E_PAD - N_EDGES,), jnp.int32)
    pad_dst = jnp.full((E_PAD - N_EDGES,), N_NODES, jnp.int32)
    srcp = jnp.concatenate([src, pad_src]).reshape(NW, NCHUNK, CHUNK)
    dstp = jnp.concatenate([dst, pad_dst]).reshape(NW, NCHUNK, CHUNK)
    ones128 = jnp.ones((CHUNK, F_HID), jnp.float32)
    zeros128 = jnp.zeros((N_ACC, F_HID), jnp.float32)

    deg = _sc_deg(dstp, ones128, zeros128)
    z1 = _tc_lin1(x, W1, deg)
    acc1 = _sc_scatter(z1, srcp, dstp, zeros128)
    z2 = _tc_lin2(acc1, z1, deg, b1.reshape(1, F_HID), W3)
    acc2 = _sc_scatter(z2, srcp, dstp, zeros128)
    return _tc_final(acc2, z2, deg, b3.reshape(1, F_HID))


# speedup vs baseline: 1.1903x; 1.1903x over previous
import functools

import jax
import jax.numpy as jnp
from jax import lax
from jax.experimental import pallas as pl
from jax.experimental.pallas import tpu as pltpu
from jax.experimental.pallas import tpu_sc as plsc

N_NODES = 10000
N_EDGES = 160000
F_IN = 256
F_HID = 128

NCORE = 2
NSUB = 16
NW = NCORE * NSUB
CHUNK = 128
EPW = 5120
NCHUNK = EPW // CHUNK
E_PAD = EPW * NW
N_ACC = 10112
ROWS_ACC = N_ACC // NSUB
ROW_BLK = 1000
NBUF = 2
NSPLIT = 2


def _vmesh():
    return plsc.VectorSubcoreMesh(core_axis_name="c", subcore_axis_name="s")


def _sc_deg(dstp, ones128, zeros128):
    @functools.partial(
        pl.kernel,
        out_type=jax.ShapeDtypeStruct((NCORE, N_ACC, F_HID), jnp.float32),
        mesh=_vmesh(),
        scratch_types=[
            pltpu.VMEM((NCHUNK, CHUNK), jnp.int32),
            pltpu.VMEM((CHUNK, F_HID), jnp.float32),
            pltpu.VMEM_SHARED((N_ACC, F_HID), jnp.float32),
        ],
    )
    def k(dst_hbm, ones_hbm, zeros_hbm, deg_hbm, dst_v, ones_v, acc_sh):
        c = lax.axis_index("c")
        s = lax.axis_index("s")
        w = c * NSUB + s
        pltpu.sync_copy(dst_hbm.at[w], dst_v)
        pltpu.sync_copy(ones_hbm, ones_v)
        pltpu.sync_copy(zeros_hbm.at[pl.ds(s * ROWS_ACC, ROWS_ACC)],
                        acc_sh.at[pl.ds(s * ROWS_ACC, ROWS_ACC)])
        plsc.subcore_barrier()

        @pl.loop(0, NCHUNK)
        def _(j):
            pltpu.sync_copy(ones_v, acc_sh.at[dst_v.at[j]], add=True)

        plsc.subcore_barrier()
        pltpu.sync_copy(acc_sh.at[pl.ds(s * ROWS_ACC, ROWS_ACC)],
                        deg_hbm.at[c, pl.ds(s * ROWS_ACC, ROWS_ACC)])

    return k(dstp, ones128, zeros128)


def _sc_scatter(z, srcp, dstp, zeros128):
    @functools.partial(
        pl.kernel,
        out_type=jax.ShapeDtypeStruct((NCORE, N_ACC, F_HID), jnp.float32),
        mesh=_vmesh(),
        scratch_types=[
            pltpu.VMEM((NCHUNK * NSPLIT, CHUNK // NSPLIT), jnp.int32),
            pltpu.VMEM((NCHUNK, CHUNK), jnp.int32),
            pltpu.VMEM((NBUF, CHUNK, F_HID), jnp.float32),
            pltpu.VMEM_SHARED((N_ACC, F_HID), jnp.float32),
            pltpu.SemaphoreType.DMA((NBUF, NSPLIT)),
        ],
    )
    def k(z_hbm, src_hbm, dst_hbm, zeros_hbm, out_hbm,
          src_v, dst_v, buf, acc_sh, gsem):
        c = lax.axis_index("c")
        s = lax.axis_index("s")
        w = c * NSUB + s
        pltpu.sync_copy(src_hbm.at[w], src_v)
        pltpu.sync_copy(dst_hbm.at[w], dst_v)
        pltpu.sync_copy(zeros_hbm.at[pl.ds(s * ROWS_ACC, ROWS_ACC)],
                        acc_sh.at[pl.ds(s * ROWS_ACC, ROWS_ACC)])
        plsc.subcore_barrier()

        part = CHUNK // NSPLIT

        def start_gathers(j, b):
            for p in range(NSPLIT):
                pltpu.async_copy(
                    z_hbm.at[src_v.at[j * NSPLIT + p]],
                    buf.at[b].at[pl.ds(p * part, part)],
                    gsem.at[b, p])

        def wait_gathers(j, b):
            for p in range(NSPLIT):
                pltpu.make_async_copy(
                    z_hbm.at[src_v.at[j * NSPLIT + p]],
                    buf.at[b].at[pl.ds(p * part, part)],
                    gsem.at[b, p]).wait()

        start_gathers(0, 0)
        start_gathers(1, 1)

        @pl.loop(0, NCHUNK, step=2)
        def _(j):
            for b in range(2):
                wait_gathers(j + b, b)
                pltpu.sync_copy(buf.at[b], acc_sh.at[dst_v.at[j + b]], add=True)

                @pl.when(j + b + 2 < NCHUNK)
                def _():
                    start_gathers(j + b + 2, b)

        plsc.subcore_barrier()
        pltpu.sync_copy(acc_sh.at[pl.ds(s * ROWS_ACC, ROWS_ACC)],
                        out_hbm.at[c, pl.ds(s * ROWS_ACC, ROWS_ACC)])

    return k(z, srcp, dstp, zeros128)


def _dinv_block(d_ref):
    d = d_ref[0][:, 0:1] + d_ref[1][:, 0:1] + 1.0
    return lax.rsqrt(d)


def _tc_lin1(x, W1, deg):
    def body(x_ref, w_ref, d_ref, o_ref):
        y = jnp.dot(x_ref[...], w_ref[...], preferred_element_type=jnp.float32)
        o_ref[...] = y * _dinv_block(d_ref)

    return pl.pallas_call(
        body,
        grid=(N_NODES // ROW_BLK,),
        in_specs=[
            pl.BlockSpec((ROW_BLK, F_IN), lambda i: (i, 0)),
            pl.BlockSpec((F_IN, F_HID), lambda i: (0, 0)),
            pl.BlockSpec((NCORE, ROW_BLK, F_HID), lambda i: (0, i, 0)),
        ],
        out_specs=pl.BlockSpec((ROW_BLK, F_HID), lambda i: (i, 0)),
        out_shape=jax.ShapeDtypeStruct((N_NODES, F_HID), jnp.float32),
    )(x, W1, deg)


def _tc_lin2(acc, z1, deg, b1, W3):
    def body(a_ref, z_ref, d_ref, b_ref, w_ref, o_ref):
        dinv = _dinv_block(d_ref)
        h = (a_ref[0] + a_ref[1] + z_ref[...]) * dinv + b_ref[...]
        h = jnp.maximum(h, 0.0)
        y = jnp.dot(h, w_ref[...], preferred_element_type=jnp.float32)
        o_ref[...] = y * dinv

    return pl.pallas_call(
        body,
        grid=(N_NODES // ROW_BLK,),
        in_specs=[
            pl.BlockSpec((NCORE, ROW_BLK, F_HID), lambda i: (0, i, 0)),
            pl.BlockSpec((ROW_BLK, F_HID), lambda i: (i, 0)),
            pl.BlockSpec((NCORE, ROW_BLK, F_HID), lambda i: (0, i, 0)),
            pl.BlockSpec((1, F_HID), lambda i: (0, 0)),
            pl.BlockSpec((F_HID, F_HID), lambda i: (0, 0)),
        ],
        out_specs=pl.BlockSpec((ROW_BLK, F_HID), lambda i: (i, 0)),
        out_shape=jax.ShapeDtypeStruct((N_NODES, F_HID), jnp.float32),
    )(acc, z1, deg, b1, W3)


def _tc_final(acc, z2, deg, b3):
    def body(a_ref, z_ref, d_ref, b_ref, o_ref):
        dinv = _dinv_block(d_ref)
        o_ref[...] = (a_ref[0] + a_ref[1] + z_ref[...]) * dinv + b_ref[...]

    return pl.pallas_call(
        body,
        grid=(N_NODES // ROW_BLK,),
        in_specs=[
            pl.BlockSpec((NCORE, ROW_BLK, F_HID), lambda i: (0, i, 0)),
            pl.BlockSpec((ROW_BLK, F_HID), lambda i: (i, 0)),
            pl.BlockSpec((NCORE, ROW_BLK, F_HID), lambda i: (0, i, 0)),
            pl.BlockSpec((1, F_HID), lambda i: (0, 0)),
        ],
        out_specs=pl.BlockSpec((ROW_BLK, F_HID), lambda i: (i, 0)),
        out_shape=jax.ShapeDtypeStruct((N_NODES, F_HID), jnp.float32),
    )(acc, z2, deg, b3)


def kernel(x, edge_index, W1, b1, W3, b3):
    src = edge_index[0]
    dst = edge_index[1]
    pad_src = jnp.zeros((E_PAD - N_EDGES,), jnp.int32)
    pad_dst = jnp.full((E_PAD - N_EDGES,), N_NODES, jnp.int32)
    srcp = jnp.concatenate([src, pad_src]).reshape(
        NW, NCHUNK * NSPLIT, CHUNK // NSPLIT)
    dstp = jnp.concatenate([dst, pad_dst]).reshape(NW, NCHUNK, CHUNK)
    ones128 = jnp.ones((CHUNK, F_HID), jnp.float32)
    zeros128 = jnp.zeros((N_ACC, F_HID), jnp.float32)

    deg = _sc_deg(dstp, ones128, zeros128)
    z1 = _tc_lin1(x, W1, deg)
    acc1 = _sc_scatter(z1, srcp, dstp, zeros128)
    z2 = _tc_lin2(acc1, z1, deg, b1.reshape(1, F_HID), W3)
    acc2 = _sc_scatter(z2, srcp, dstp, zeros128)
    return _tc_final(acc2, z2, deg, b3.reshape(1, F_HID))


# speedup vs baseline: 1.2969x; 1.0896x over previous
import functools

import jax
import jax.numpy as jnp
from jax import lax
from jax.experimental import pallas as pl
from jax.experimental.pallas import tpu as pltpu
from jax.experimental.pallas import tpu_sc as plsc

N_NODES = 10000
N_EDGES = 160000
F_IN = 256
F_HID = 128

NCORE = 2
NSUB = 16
NW = NCORE * NSUB
CHUNK = 128
EPW = 5120
NCHUNK = EPW // CHUNK
E_PAD = EPW * NW
N_ACC = 10112
ROWS_ACC = N_ACC // NSUB
ROW_BLK = 1000
NBUF = 2
NC0 = 60
NC1 = 20


def _vmesh():
    return plsc.VectorSubcoreMesh(core_axis_name="c", subcore_axis_name="s")


def _sc_deg(dstp, ones128, zeros128):
    @functools.partial(
        pl.kernel,
        out_type=jax.ShapeDtypeStruct((NCORE, N_ACC, F_HID), jnp.float32),
        mesh=_vmesh(),
        scratch_types=[
            pltpu.VMEM((NCHUNK, CHUNK), jnp.int32),
            pltpu.VMEM((CHUNK, F_HID), jnp.float32),
            pltpu.VMEM_SHARED((N_ACC, F_HID), jnp.float32),
        ],
    )
    def k(dst_hbm, ones_hbm, zeros_hbm, deg_hbm, dst_v, ones_v, acc_sh):
        c = lax.axis_index("c")
        s = lax.axis_index("s")
        w = c * NSUB + s
        pltpu.sync_copy(dst_hbm.at[w], dst_v)
        pltpu.sync_copy(ones_hbm, ones_v)
        pltpu.sync_copy(zeros_hbm.at[pl.ds(s * ROWS_ACC, ROWS_ACC)],
                        acc_sh.at[pl.ds(s * ROWS_ACC, ROWS_ACC)])
        plsc.subcore_barrier()

        @pl.loop(0, NCHUNK)
        def _(j):
            pltpu.sync_copy(ones_v, acc_sh.at[dst_v.at[j]], add=True)

        plsc.subcore_barrier()
        pltpu.sync_copy(acc_sh.at[pl.ds(s * ROWS_ACC, ROWS_ACC)],
                        deg_hbm.at[c, pl.ds(s * ROWS_ACC, ROWS_ACC)])

    return k(dstp, ones128, zeros128)


def _sc_scatter(z, srcp, dstp, zeros128):
    @functools.partial(
        pl.kernel,
        out_type=jax.ShapeDtypeStruct((NCORE, N_ACC, F_HID), jnp.float32),
        mesh=_vmesh(),
        scratch_types=[
            pltpu.VMEM((NC0, CHUNK), jnp.int32),
            pltpu.VMEM((NC0, CHUNK), jnp.int32),
            pltpu.VMEM((NBUF, CHUNK, F_HID), jnp.float32),
            pltpu.VMEM_SHARED((N_ACC, F_HID), jnp.float32),
            pltpu.SemaphoreType.DMA((NBUF,)),
        ],
    )
    def k(z_hbm, src_hbm, dst_hbm, zeros_hbm, out_hbm,
          src_v, dst_v, buf, acc_sh, gsem):
        c = lax.axis_index("c")
        s = lax.axis_index("s")
        w = c * NSUB + s
        nc = lax.select(c == 0, NC0, NC1)
        pltpu.sync_copy(src_hbm.at[w], src_v)
        pltpu.sync_copy(dst_hbm.at[w], dst_v)
        pltpu.sync_copy(zeros_hbm.at[pl.ds(s * ROWS_ACC, ROWS_ACC)],
                        acc_sh.at[pl.ds(s * ROWS_ACC, ROWS_ACC)])
        plsc.subcore_barrier()

        pltpu.async_copy(z_hbm.at[src_v.at[0]], buf.at[0], gsem.at[0])
        pltpu.async_copy(z_hbm.at[src_v.at[1]], buf.at[1], gsem.at[1])

        @pl.loop(0, nc, step=2)
        def _(j):
            for b in range(2):
                pltpu.make_async_copy(z_hbm.at[src_v.at[j + b]], buf.at[b],
                                      gsem.at[b]).wait()
                pltpu.sync_copy(buf.at[b], acc_sh.at[dst_v.at[j + b]], add=True)

                @pl.when(j + b + 2 < nc)
                def _():
                    pltpu.async_copy(z_hbm.at[src_v.at[j + b + 2]],
                                     buf.at[b], gsem.at[b])

        plsc.subcore_barrier()
        pltpu.sync_copy(acc_sh.at[pl.ds(s * ROWS_ACC, ROWS_ACC)],
                        out_hbm.at[c, pl.ds(s * ROWS_ACC, ROWS_ACC)])

    return k(z, srcp, dstp, zeros128)


def _dinv_block(d_ref):
    d = d_ref[0][:, 0:1] + d_ref[1][:, 0:1] + 1.0
    return lax.rsqrt(d)


def _tc_lin1(x, W1, deg):
    def body(x_ref, w_ref, d_ref, o_ref):
        y = jnp.dot(x_ref[...], w_ref[...], preferred_element_type=jnp.float32)
        o_ref[...] = y * _dinv_block(d_ref)

    return pl.pallas_call(
        body,
        grid=(N_NODES // ROW_BLK,),
        in_specs=[
            pl.BlockSpec((ROW_BLK, F_IN), lambda i: (i, 0)),
            pl.BlockSpec((F_IN, F_HID), lambda i: (0, 0)),
            pl.BlockSpec((NCORE, ROW_BLK, F_HID), lambda i: (0, i, 0)),
        ],
        out_specs=pl.BlockSpec((ROW_BLK, F_HID), lambda i: (i, 0)),
        out_shape=jax.ShapeDtypeStruct((N_NODES, F_HID), jnp.float32),
    )(x, W1, deg)


def _tc_lin2(acc, z1, deg, b1, W3):
    def body(a_ref, z_ref, d_ref, b_ref, w_ref, o_ref):
        dinv = _dinv_block(d_ref)
        h = (a_ref[0] + a_ref[1] + z_ref[...]) * dinv + b_ref[...]
        h = jnp.maximum(h, 0.0)
        y = jnp.dot(h, w_ref[...], preferred_element_type=jnp.float32)
        o_ref[...] = y * dinv

    return pl.pallas_call(
        body,
        grid=(N_NODES // ROW_BLK,),
        in_specs=[
            pl.BlockSpec((NCORE, ROW_BLK, F_HID), lambda i: (0, i, 0)),
            pl.BlockSpec((ROW_BLK, F_HID), lambda i: (i, 0)),
            pl.BlockSpec((NCORE, ROW_BLK, F_HID), lambda i: (0, i, 0)),
            pl.BlockSpec((1, F_HID), lambda i: (0, 0)),
            pl.BlockSpec((F_HID, F_HID), lambda i: (0, 0)),
        ],
        out_specs=pl.BlockSpec((ROW_BLK, F_HID), lambda i: (i, 0)),
        out_shape=jax.ShapeDtypeStruct((N_NODES, F_HID), jnp.float32),
    )(acc, z1, deg, b1, W3)


def _tc_final(acc, z2, deg, b3):
    def body(a_ref, z_ref, d_ref, b_ref, o_ref):
        dinv = _dinv_block(d_ref)
        o_ref[...] = (a_ref[0] + a_ref[1] + z_ref[...]) * dinv + b_ref[...]

    return pl.pallas_call(
        body,
        grid=(N_NODES // ROW_BLK,),
        in_specs=[
            pl.BlockSpec((NCORE, ROW_BLK, F_HID), lambda i: (0, i, 0)),
            pl.BlockSpec((ROW_BLK, F_HID), lambda i: (i, 0)),
            pl.BlockSpec((NCORE, ROW_BLK, F_HID), lambda i: (0, i, 0)),
            pl.BlockSpec((1, F_HID), lambda i: (0, 0)),
        ],
        out_specs=pl.BlockSpec((ROW_BLK, F_HID), lambda i: (i, 0)),
        out_shape=jax.ShapeDtypeStruct((N_NODES, F_HID), jnp.float32),
    )(acc, z2, deg, b3)


def kernel(x, edge_index, W1, b1, W3, b3):
    src = edge_index[0]
    dst = edge_index[1]
    pad_src = jnp.zeros((E_PAD - N_EDGES,), jnp.int32)
    pad_dst = jnp.full((E_PAD - N_EDGES,), N_NODES, jnp.int32)
    dstp = jnp.concatenate([dst, pad_dst]).reshape(NW, NCHUNK, CHUNK)

    def asym(idx, pad_val):
        e0 = NSUB * NC0 * CHUNK
        e1cap = NSUB * NC1 * CHUNK
        a = idx[:e0].reshape(NSUB, NC0, CHUNK)
        b = jnp.concatenate(
            [idx[e0:], jnp.full((e0 + e1cap - N_EDGES,), pad_val, jnp.int32)]
        ).reshape(NSUB, NC1, CHUNK)
        b = jnp.concatenate(
            [b, jnp.full((NSUB, NC0 - NC1, CHUNK), pad_val, jnp.int32)], axis=1)
        return jnp.concatenate([a, b], axis=0)

    srcp = asym(src, 0)
    dstp_s = asym(dst, N_NODES)
    ones128 = jnp.ones((CHUNK, F_HID), jnp.float32)
    zeros128 = jnp.zeros((N_ACC, F_HID), jnp.float32)

    deg = _sc_deg(dstp, ones128, zeros128)
    z1 = _tc_lin1(x, W1, deg)
    acc1 = _sc_scatter(z1, srcp, dstp_s, zeros128)
    z2 = _tc_lin2(acc1, z1, deg, b1.reshape(1, F_HID), W3)
    acc2 = _sc_scatter(z2, srcp, dstp_s, zeros128)
    return _tc_final(acc2, z2, deg, b3.reshape(1, F_HID))
